# baseline (speedup 1.0000x reference)
import math

import jax
import jax.numpy as jnp
from jax.experimental import pallas as pl
from jax.experimental.pallas import tpu as pltpu

_BLUR = 0.05
_EPS = _BLUR ** 2
_INV_EPS = 1.0 / _EPS
_N_ITERS = 10
_LN2 = math.log(2.0)
_L2E = 1.0 / _LN2


def kernel(y_pred, y_true, length_pred, length_true):
    B, T = y_pred.shape
    Tm2 = T - 2
    BLK = 128
    SB = 2
    COLS = SB * BLK
    G = B // COLS
    NCH = 3 * SB
    log_a = math.log(1.0 / T)

    def _body(ypt_ref, ytt_ref, lp_ref, lt_ref, out_ref, *trefs):
        c2 = 0.5 * _INV_EPS * _L2E
        sc = math.sqrt(c2)
        la2 = log_a * _L2E

        def pass1(F, Cm, t_ref):
            t = F[0, None, :] - Cm[0]
            t_ref[0] = t
            m = t
            for i in range(1, Tm2):
                t = F[i, None, :] - Cm[i]
                t_ref[i] = t
                m = jnp.maximum(m, t)
            return m

        def pass2(m, t_ref):
            s = jnp.exp2(t_ref[0] - m)
            for i in range(1, Tm2):
                s = s + jnp.exp2(t_ref[i] - m)
            return m + jnp.log2(s)

        def lse_zero(Cm):
            mn = Cm[0]
            for i in range(1, Tm2):
                mn = jnp.minimum(mn, Cm[i])
            s = jnp.exp2(mn - Cm[0])
            for i in range(1, Tm2):
                s = s + jnp.exp2(mn - Cm[i])
            return -mn + jnp.log2(s)

        mats = []
        yps, yts = [], []
        for sb in range(SB):
            lo, hi = sb * BLK, (sb + 1) * BLK
            yp = ypt_ref[1:T - 1, lo:hi]
            yt = ytt_ref[1:T - 1, lo:hi]
            yps.append(yp)
            yts.append(yt)
            xs = yt * sc
            ys = yp * sc
            Ce0 = (xs[:, None, :] - ys[None, :, :]) ** 2
            Ce0T = (ys[:, None, :] - xs[None, :, :]) ** 2
            Ce1 = (xs[:, None, :] - xs[None, :, :]) ** 2
            Ce2 = (ys[:, None, :] - ys[None, :, :]) ** 2
            mats += [(Ce0, Ce0T), (Ce1, Ce1), (Ce2, Ce2)]

        z = jnp.zeros((Tm2, BLK), jnp.float32)
        cur = [z] * NCH
        prev = [z] * NCH
        m_pend = [None] * NCH
        n_ops = 4 * _N_ITERS

        def chain_op(q, idx):
            k, phase = divmod(idx, 2)
            if k == 0:
                if phase == 0:
                    prev[q] = cur[q]
                    cur[q] = -(lse_zero(mats[q][0]) + la2)
                return
            if phase == 0:
                m_pend[q] = pass1(cur[q], mats[q][k % 2], trefs[q])
            else:
                prev[q] = cur[q]
                cur[q] = -(pass2(m_pend[q], trefs[q]) + la2)

        for t_slot in range(n_ops + NCH - 1):
            for q in range(NCH):
                idx = t_slot - q
                if 0 <= idx < n_ops:
                    chain_op(q, idx)

        w = _EPS * _LN2 / T
        divs, tims, dl2s = [], [], []
        for sb in range(SB):
            ots = [(jnp.sum(cur[3 * sb + p], axis=0)
                    + jnp.sum(prev[3 * sb + p], axis=0)) * w
                   for p in range(3)]
            divs.append(ots[0] - 0.5 * ots[1] - 0.5 * ots[2])
            tims.append(jnp.sum((yps[sb] - yts[sb]) ** 2, axis=0))
            lo, hi = sb * BLK, (sb + 1) * BLK
            dl = lp_ref[0, lo:hi] - lt_ref[0, lo:hi]
            dl2s.append(dl * dl)
        out_ref[0] = jnp.stack([
            jnp.concatenate(divs), jnp.concatenate(tims),
            jnp.concatenate(dl2s)])

    out = pl.pallas_call(
        _body,
        grid=(G,),
        in_specs=[
            pl.BlockSpec((T, COLS), lambda i: (0, i)),
            pl.BlockSpec((T, COLS), lambda i: (0, i)),
            pl.BlockSpec((1, COLS), lambda i: (0, i)),
            pl.BlockSpec((1, COLS), lambda i: (0, i)),
        ],
        out_specs=pl.BlockSpec((1, 3, COLS), lambda i: (i, 0, 0)),
        out_shape=jax.ShapeDtypeStruct((G, 3, COLS), jnp.float32),
        scratch_shapes=[pltpu.VMEM((Tm2, Tm2, BLK), jnp.float32)] * NCH,
        compiler_params=pltpu.CompilerParams(
            dimension_semantics=("parallel",)),
    )(y_pred.T, y_true.T, length_pred.reshape(1, B), length_true.reshape(1, B))

    sums = out.sum(axis=(0, 2))
    distrib_loss = sums[0] / B
    timing_loss = sums[1] / (B * Tm2)
    length_loss = sums[2] / B
    weighted_loss = timing_loss + length_loss + distrib_loss
    return (weighted_loss, length_loss, timing_loss)


# speedup vs baseline: 1.2219x; 1.2219x over previous
import math

import jax
import jax.numpy as jnp
from jax.experimental import pallas as pl
from jax.experimental.pallas import tpu as pltpu

_BLUR = 0.05
_EPS = _BLUR ** 2
_INV_EPS = 1.0 / _EPS
_N_ITERS = 10
_LN2 = math.log(2.0)
_L2E = 1.0 / _LN2


def kernel(y_pred, y_true, length_pred, length_true):
    B, T = y_pred.shape
    Tm2 = T - 2
    BLK = 128
    G = B // BLK
    log_a = math.log(1.0 / T)

    def _body(ypt_ref, ytt_ref, lp_ref, lt_ref, out_ref, t0_ref, t1_ref,
              t2_ref):
        yp = ypt_ref[1:T - 1, :]
        yt = ytt_ref[1:T - 1, :]
        c2 = 0.5 * _INV_EPS * _L2E
        sc = math.sqrt(c2)
        xs = yt * sc
        ys = yp * sc
        Ce0 = (xs[:, None, :] - ys[None, :, :]) ** 2
        Ce0T = (ys[:, None, :] - xs[None, :, :]) ** 2
        Ce1 = (xs[:, None, :] - xs[None, :, :]) ** 2
        Ce2 = (ys[:, None, :] - ys[None, :, :]) ** 2

        la2 = log_a * _L2E

        def pass1(F, Cm, t_ref):
            t = F[0, None, :] - Cm[0]
            t_ref[0] = t
            m = t
            for i in range(1, Tm2):
                t = F[i, None, :] - Cm[i]
                t_ref[i] = t
                m = jnp.maximum(m, t)
            return m

        def pass2(m, t_ref):
            s = jnp.exp2(t_ref[0] - m)
            for i in range(1, Tm2):
                s = s + jnp.exp2(t_ref[i] - m)
            return m + jnp.log2(s)

        def lse_zero(Cm):
            mn = Cm[0]
            for i in range(1, Tm2):
                mn = jnp.minimum(mn, Cm[i])
            s = jnp.exp2(mn - Cm[0])
            for i in range(1, Tm2):
                s = s + jnp.exp2(mn - Cm[i])
            return -mn + jnp.log2(s)

        z = jnp.zeros((Tm2, BLK), jnp.float32)
        mats = [(Ce0, Ce0T), (Ce1, Ce1), (Ce2, Ce2)]
        trefs = (t0_ref, t1_ref, t2_ref)
        cur = [z, z, z]
        prev = [z, z, z]
        m_pend = [None, None, None]
        n_ops = 4 * _N_ITERS

        def chain_op(p, idx):
            k, phase = divmod(idx, 2)
            if k == 0:
                if phase == 0:
                    prev[p] = cur[p]
                    cur[p] = -(lse_zero(mats[p][0]) + la2)
                return
            if phase == 0:
                m_pend[p] = pass1(cur[p], mats[p][k % 2], trefs[p])
            else:
                prev[p] = cur[p]
                cur[p] = -(pass2(m_pend[p], trefs[p]) + la2)

        for t_slot in range(n_ops + 2):
            for p in range(3):
                idx = t_slot - p
                if 0 <= idx < n_ops:
                    chain_op(p, idx)

        w = _EPS * _LN2 / T
        ot0 = (jnp.sum(cur[0], axis=0) + jnp.sum(prev[0], axis=0)) * w
        ot1 = (jnp.sum(cur[1], axis=0) + jnp.sum(prev[1], axis=0)) * w
        ot2 = (jnp.sum(cur[2], axis=0) + jnp.sum(prev[2], axis=0)) * w
        div = ot0 - 0.5 * ot1 - 0.5 * ot2
        tim = jnp.sum((yp - yt) ** 2, axis=0)
        dl = lp_ref[0, :] - lt_ref[0, :]
        out_ref[0] = jnp.stack([div, tim, dl * dl])

    out = pl.pallas_call(
        _body,
        grid=(G,),
        in_specs=[
            pl.BlockSpec((T, BLK), lambda i: (0, i)),
            pl.BlockSpec((T, BLK), lambda i: (0, i)),
            pl.BlockSpec((1, BLK), lambda i: (0, i)),
            pl.BlockSpec((1, BLK), lambda i: (0, i)),
        ],
        out_specs=pl.BlockSpec((1, 3, BLK), lambda i: (i, 0, 0)),
        out_shape=jax.ShapeDtypeStruct((G, 3, BLK), jnp.float32),
        scratch_shapes=[pltpu.VMEM((Tm2, Tm2, BLK), jnp.float32)] * 3,
        compiler_params=pltpu.CompilerParams(
            dimension_semantics=("parallel",)),
    )(y_pred.T, y_true.T, length_pred.reshape(1, B), length_true.reshape(1, B))

    sums = out.sum(axis=(0, 2))
    distrib_loss = sums[0] / B
    timing_loss = sums[1] / (B * Tm2)
    length_loss = sums[2] / B
    weighted_loss = timing_loss + length_loss + distrib_loss
    return (weighted_loss, length_loss, timing_loss)


# speedup vs baseline: 1.2315x; 1.0079x over previous
import math

import jax
import jax.numpy as jnp
from jax.experimental import pallas as pl
from jax.experimental.pallas import tpu as pltpu

_BLUR = 0.05
_EPS = _BLUR ** 2
_INV_EPS = 1.0 / _EPS
_N_ITERS = 10
_LN2 = math.log(2.0)
_L2E = 1.0 / _LN2


def kernel(y_pred, y_true, length_pred, length_true):
    B, T = y_pred.shape
    Tm2 = T - 2
    BLK = 128
    G = B // BLK
    log_a = math.log(1.0 / T)

    def _body(ypt_ref, ytt_ref, lp_ref, lt_ref, out_ref, t0_ref, t1_ref,
              t2_ref):
        yp = ypt_ref[1:T - 1, :]
        yt = ytt_ref[1:T - 1, :]
        c2 = 0.5 * _INV_EPS * _L2E
        sc = math.sqrt(c2)
        xs = yt * sc
        ys = yp * sc
        Ce0 = (xs[:, None, :] - ys[None, :, :]) ** 2
        Ce0T = (ys[:, None, :] - xs[None, :, :]) ** 2
        Ce1 = (xs[:, None, :] - xs[None, :, :]) ** 2
        Ce2 = (ys[:, None, :] - ys[None, :, :]) ** 2

        la2 = log_a * _L2E

        def pass1(F, Cm, t_ref):
            t = F[0, None, :] - Cm[0]
            t_ref[0] = t
            m = t
            for i in range(1, Tm2):
                t = F[i, None, :] - Cm[i]
                t_ref[i] = t
                m = jnp.maximum(m, t)
            return m

        def pass2(m, t_ref):
            s = jnp.exp2(t_ref[0] - m)
            for i in range(1, Tm2):
                s = s + jnp.exp2(t_ref[i] - m)
            return m + jnp.log2(s)

        def lse_zero(Cm):
            mn = Cm[0]
            for i in range(1, Tm2):
                mn = jnp.minimum(mn, Cm[i])
            s = jnp.exp2(mn - Cm[0])
            for i in range(1, Tm2):
                s = s + jnp.exp2(mn - Cm[i])
            return -mn + jnp.log2(s)

        z = jnp.zeros((Tm2, BLK), jnp.float32)
        mats = [(Ce0, Ce0T), (Ce1, Ce1), (Ce2, Ce2)]
        trefs = (t0_ref, t1_ref, t2_ref)
        cur = [z, z, z]
        prev = [z, z, z]
        m_pend = [None, None, None]
        n_ops = 4 * _N_ITERS

        def chain_op(p, idx):
            k, phase = divmod(idx, 2)
            if k == 0:
                if phase == 0:
                    prev[p] = cur[p]
                    cur[p] = -(lse_zero(mats[p][0]) + la2)
                return
            if phase == 0:
                m_pend[p] = pass1(cur[p], mats[p][k % 2], trefs[p])
            else:
                prev[p] = cur[p]
                cur[p] = -(pass2(m_pend[p], trefs[p]) + la2)

        for t_slot in range(n_ops + 4):
            for p in range(3):
                idx = t_slot - 2 * p
                if 0 <= idx < n_ops:
                    chain_op(p, idx)

        w = _EPS * _LN2 / T
        ot0 = (jnp.sum(cur[0], axis=0) + jnp.sum(prev[0], axis=0)) * w
        ot1 = (jnp.sum(cur[1], axis=0) + jnp.sum(prev[1], axis=0)) * w
        ot2 = (jnp.sum(cur[2], axis=0) + jnp.sum(prev[2], axis=0)) * w
        div = ot0 - 0.5 * ot1 - 0.5 * ot2
        tim = jnp.sum((yp - yt) ** 2, axis=0)
        dl = lp_ref[0, :] - lt_ref[0, :]
        out_ref[0] = jnp.stack([div, tim, dl * dl])

    out = pl.pallas_call(
        _body,
        grid=(G,),
        in_specs=[
            pl.BlockSpec((T, BLK), lambda i: (0, i)),
            pl.BlockSpec((T, BLK), lambda i: (0, i)),
            pl.BlockSpec((1, BLK), lambda i: (0, i)),
            pl.BlockSpec((1, BLK), lambda i: (0, i)),
        ],
        out_specs=pl.BlockSpec((1, 3, BLK), lambda i: (i, 0, 0)),
        out_shape=jax.ShapeDtypeStruct((G, 3, BLK), jnp.float32),
        scratch_shapes=[pltpu.VMEM((Tm2, Tm2, BLK), jnp.float32)] * 3,
        compiler_params=pltpu.CompilerParams(
            dimension_semantics=("parallel",)),
    )(y_pred.T, y_true.T, length_pred.reshape(1, B), length_true.reshape(1, B))

    sums = out.sum(axis=(0, 2))
    distrib_loss = sums[0] / B
    timing_loss = sums[1] / (B * Tm2)
    length_loss = sums[2] / B
    weighted_loss = timing_loss + length_loss + distrib_loss
    return (weighted_loss, length_loss, timing_loss)
